# Initial kernel scaffold; baseline (speedup 1.0000x reference)
import jax
import jax.numpy as jnp
from jax.experimental import pallas as pl


def kernel(x, edge_index, edge_type, comp1, basis1, root1, bias1, comp2, basis2, root2, bias2):
    raise NotImplementedError("write your pallas kernel here")


# speedup vs baseline: 19.3153x; 19.3153x over previous
import functools
import jax
import jax.numpy as jnp
from jax import lax
from jax.experimental import pallas as pl
from jax.experimental.pallas import tpu as pltpu
from jax.experimental.pallas import tpu_sc as plsc

N = 10000
E = 320000
IN_CH = 128
HID = 64
OUT_CH = 128
R = 8

NC = 2
NS = 16
NW = NC * NS
L = 16

CHUNK = 128
TE = 10112
NCHUNK = TE // CHUNK
E_PAD = TE * NW
RN = R * N
RN_SZ = 80128
CSLICE = RN_SZ // NS
N_PAD = 10240
NROW = N_PAD // NS
ZROW = 128

_mesh = plsc.VectorSubcoreMesh(core_axis_name="c", subcore_axis_name="s")


def _fori(lo, hi, body, unroll=1):
    lax.fori_loop(lo, hi, lambda i, c: (body(i), c)[1], 0, unroll=unroll)


@functools.partial(
    pl.kernel,
    out_type=jax.ShapeDtypeStruct((NC * RN_SZ,), jnp.float32),
    mesh=_mesh,
    compiler_params=pltpu.CompilerParams(needs_layout_passes=False),
    scratch_types=[
        pltpu.VMEM((CHUNK,), jnp.int32),
        pltpu.VMEM((CHUNK,), jnp.float32),
        pltpu.VMEM((CSLICE,), jnp.float32),
        pltpu.VMEM_SHARED((RN_SZ,), jnp.float32),
    ],
)
def _counts_kernel(cidx_hbm, out_hbm, cidx_v, ones_v, zbuf, counts_sh):
    c = lax.axis_index("c")
    s = lax.axis_index("s")
    wid = c * NS + s
    zero16 = jnp.zeros((L,), jnp.float32)
    one16 = jnp.ones((L,), jnp.float32)
    _fori(0, CSLICE // L, lambda i: zbuf.__setitem__(pl.ds(i * L, L), zero16))
    _fori(0, CHUNK // L, lambda i: ones_v.__setitem__(pl.ds(i * L, L), one16))
    pltpu.sync_copy(zbuf, counts_sh.at[pl.ds(s * CSLICE, CSLICE)])
    plsc.subcore_barrier()

    def chunk(i):
        base = wid * TE + i * CHUNK
        pltpu.sync_copy(cidx_hbm.at[pl.ds(base, CHUNK)], cidx_v)
        pltpu.sync_copy(ones_v, counts_sh.at[cidx_v], add=True)

    _fori(0, NCHUNK, chunk)
    plsc.subcore_barrier()
    pltpu.sync_copy(counts_sh.at[pl.ds(s * CSLICE, CSLICE)], zbuf)
    pltpu.sync_copy(zbuf, out_hbm.at[pl.ds(c * RN_SZ + s * CSLICE, CSLICE)])


@functools.partial(
    pl.kernel,
    out_type=jax.ShapeDtypeStruct((E_PAD,), jnp.float32),
    mesh=_mesh,
    compiler_params=pltpu.CompilerParams(needs_layout_passes=False),
    scratch_types=[
        pltpu.VMEM((RN_SZ,), jnp.float32),
        pltpu.VMEM((CSLICE,), jnp.float32),
        pltpu.VMEM((CHUNK,), jnp.int32),
        pltpu.VMEM((CHUNK,), jnp.float32),
    ],
)
def _weights_kernel(parts_hbm, cidx_hbm, w_hbm, counts_f, tmp, cidx_v, w_v):
    c = lax.axis_index("c")
    s = lax.axis_index("s")
    wid = c * NS + s
    pltpu.sync_copy(parts_hbm.at[pl.ds(0, RN_SZ)], counts_f)

    def addslice(k):
        pltpu.sync_copy(parts_hbm.at[pl.ds(RN_SZ + k * CSLICE, CSLICE)], tmp)

        def addv(i):
            sl = pl.ds(k * CSLICE + i * L, L)
            counts_f[sl] = counts_f[sl] + tmp[pl.ds(i * L, L)]

        _fori(0, CSLICE // L, addv, unroll=4)

    _fori(0, NS, addslice)

    iota = lax.iota(jnp.int32, L)

    def chunk(i):
        base = wid * TE + i * CHUNK
        pltpu.sync_copy(cidx_hbm.at[pl.ds(base, CHUNK)], cidx_v)

        def step(j):
            civ = cidx_v[pl.ds(j * L, L)]
            cnt = plsc.load_gather(counts_f, [civ])
            w = 1.0 / jnp.maximum(cnt, 1.0)
            eid = base + j * L + iota
            w_v[pl.ds(j * L, L)] = jnp.where(eid < E, w, 0.0)

        _fori(0, CHUNK // L, step)
        pltpu.sync_copy(w_v, w_hbm.at[pl.ds(base, CHUNK)])

    _fori(0, NCHUNK, chunk)


def _make_msg_kernel(out_ch):
    @functools.partial(
        pl.kernel,
        out_type=jax.ShapeDtypeStruct((NC * N_PAD, out_ch), jnp.float32),
        mesh=_mesh,
        compiler_params=pltpu.CompilerParams(
            needs_layout_passes=False, use_tc_tiling_on_sc=False),
        scratch_types=[
            pltpu.VMEM((CHUNK,), jnp.int32),
            pltpu.VMEM((CHUNK,), jnp.int32),
            pltpu.VMEM((CHUNK,), jnp.float32),
            pltpu.VMEM((CHUNK, out_ch), jnp.float32),
            pltpu.VMEM((ZROW, out_ch), jnp.float32),
            pltpu.VMEM_SHARED((N_PAD, out_ch), jnp.float32),
            pltpu.SemaphoreType.DMA,
        ],
    )
    def _msg_kernel(hx_hbm, gidx_hbm, dst_hbm, w_hbm, out_hbm,
                    gidx_v, dst_v, w_v, rows_v, zbuf, acc, sem):
        c = lax.axis_index("c")
        s = lax.axis_index("s")
        wid = c * NS + s
        zero16 = jnp.zeros((L,), jnp.float32)

        def zrow(i):
            for t in range(out_ch // L):
                zbuf[i, pl.ds(t * L, L)] = zero16

        _fori(0, ZROW, zrow)
        for k in range(NROW // ZROW):
            pltpu.sync_copy(zbuf, acc.at[pl.ds(s * NROW + k * ZROW, ZROW)])
        plsc.subcore_barrier()

        def chunk(i):
            base = wid * TE + i * CHUNK
            pltpu.sync_copy(gidx_hbm.at[pl.ds(base, CHUNK)], gidx_v)
            pltpu.sync_copy(dst_hbm.at[pl.ds(base, CHUNK)], dst_v)
            pltpu.sync_copy(w_hbm.at[pl.ds(base, CHUNK)], w_v)
            pltpu.async_copy(hx_hbm.at[gidx_v], rows_v, sem).wait()

            def srow(j):
                wb = plsc.load_gather(w_v, [jnp.full((L,), j, jnp.int32)])
                for t in range(out_ch // L):
                    sl = pl.ds(t * L, L)
                    rows_v[j, sl] = rows_v[j, sl] * wb

            _fori(0, CHUNK, srow, unroll=2)
            pltpu.sync_copy(rows_v, acc.at[dst_v], add=True)

        _fori(0, NCHUNK, chunk)
        plsc.subcore_barrier()
        for k in range(NROW // ZROW):
            sl = pl.ds(s * NROW + k * ZROW, ZROW)
            pltpu.sync_copy(acc.at[sl], zbuf)
            pltpu.sync_copy(zbuf, out_hbm.at[pl.ds(c * N_PAD + s * NROW + k * ZROW, ZROW)])

    return _msg_kernel


_msg_kernel_l1 = _make_msg_kernel(HID)
_msg_kernel_l2 = _make_msg_kernel(OUT_CH)


_BN = 1000


def _mm1_body(comp_ref, basis_ref, root_ref, bias_ref, x_ref, hx_ref, self_ref):
    xb = x_ref[...]
    ws = []
    for r in range(R):
        w = comp_ref[r, 0] * basis_ref[0]
        for b in range(1, 4):
            w = w + comp_ref[r, b] * basis_ref[b]
        ws.append(w)
    wcat = jnp.concatenate(ws, axis=1)
    hx_ref[...] = jnp.dot(xb, wcat, preferred_element_type=jnp.float32)
    self_ref[...] = (jnp.dot(xb, root_ref[...], preferred_element_type=jnp.float32)
                     + bias_ref[...])


def _mm2_body(comp_ref, basis_ref, root_ref, bias_ref, self1_ref, p0_ref, p1_ref,
              hx_ref, self_ref):
    h = jnp.maximum(self1_ref[...] + p0_ref[...] + p1_ref[...], 0.0)
    ws = []
    for r in range(R):
        w = comp_ref[r, 0] * basis_ref[0]
        for b in range(1, 4):
            w = w + comp_ref[r, b] * basis_ref[b]
        ws.append(w)
    wcat = jnp.concatenate(ws, axis=1)
    hx_ref[...] = jnp.dot(h, wcat, preferred_element_type=jnp.float32)
    self_ref[...] = (jnp.dot(h, root_ref[...], preferred_element_type=jnp.float32)
                     + bias_ref[...])


def _final_body(a_ref, b_ref, c_ref, o_ref):
    o_ref[...] = a_ref[...] + b_ref[...] + c_ref[...]


def _full_spec(shape):
    nd = len(shape)
    return pl.BlockSpec(shape, lambda i, _n=nd: (0,) * _n)


def _row_spec(cols):
    return pl.BlockSpec((_BN, cols), lambda i: (i, 0))


def _mm1(x, comp, basis, root, bias):
    return pl.pallas_call(
        _mm1_body,
        grid=(N // _BN,),
        in_specs=[
            pl.BlockSpec(memory_space=pltpu.SMEM),
            _full_spec((4, IN_CH, HID)),
            _full_spec((IN_CH, HID)),
            _full_spec((1, HID)),
            _row_spec(IN_CH),
        ],
        out_specs=[_row_spec(R * HID), _row_spec(HID)],
        out_shape=[
            jax.ShapeDtypeStruct((N, R * HID), jnp.float32),
            jax.ShapeDtypeStruct((N, HID), jnp.float32),
        ],
    )(comp, basis, root, bias.reshape(1, HID), x)


def _mm2(self1, p0, p1, comp, basis, root, bias):
    return pl.pallas_call(
        _mm2_body,
        grid=(N // _BN,),
        in_specs=[
            pl.BlockSpec(memory_space=pltpu.SMEM),
            _full_spec((4, HID, OUT_CH)),
            _full_spec((HID, OUT_CH)),
            _full_spec((1, OUT_CH)),
            _row_spec(HID),
            _row_spec(HID),
            _row_spec(HID),
        ],
        out_specs=[_row_spec(R * OUT_CH), _row_spec(OUT_CH)],
        out_shape=[
            jax.ShapeDtypeStruct((N, R * OUT_CH), jnp.float32),
            jax.ShapeDtypeStruct((N, OUT_CH), jnp.float32),
        ],
    )(comp, basis, root, bias.reshape(1, OUT_CH), self1, p0, p1)


def _final(a, b, c):
    return pl.pallas_call(
        _final_body,
        grid=(N // _BN,),
        in_specs=[_row_spec(OUT_CH)] * 3,
        out_specs=_row_spec(OUT_CH),
        out_shape=jax.ShapeDtypeStruct((N, OUT_CH), jnp.float32),
    )(a, b, c)


def kernel(x, edge_index, edge_type, comp1, basis1, root1, bias1,
           comp2, basis2, root2, bias2):
    src = edge_index[0].astype(jnp.int32)
    dst = edge_index[1].astype(jnp.int32)
    et = edge_type.astype(jnp.int32)
    gidx = src * R + et
    cidx = dst * R + et
    pad = E_PAD - E
    gidx_p = jnp.concatenate([gidx, jnp.zeros((pad,), jnp.int32)])
    dst_p = jnp.concatenate([dst, jnp.zeros((pad,), jnp.int32)])
    cidx_p = jnp.concatenate([cidx, jnp.full((pad,), RN, jnp.int32)])

    parts = _counts_kernel(cidx_p)
    w = _weights_kernel(parts, cidx_p)

    hx1, self1 = _mm1(x, comp1, basis1, root1, bias1)
    p = _msg_kernel_l1(hx1.reshape(N * R, HID), gidx_p, dst_p, w)
    hx2, self2 = _mm2(self1, p[:N], p[N_PAD:N_PAD + N], comp2, basis2, root2, bias2)
    q = _msg_kernel_l2(hx2.reshape(N * R, OUT_CH), gidx_p, dst_p, w)
    return _final(self2, q[:N], q[N_PAD:N_PAD + N])


# speedup vs baseline: 24.6993x; 1.2787x over previous
import functools
import jax
import jax.numpy as jnp
from jax import lax
from jax.experimental import pallas as pl
from jax.experimental.pallas import tpu as pltpu
from jax.experimental.pallas import tpu_sc as plsc

N = 10000
E = 320000
IN_CH = 128
HID = 64
OUT_CH = 128
R = 8

NC = 2
NS = 16
NW = NC * NS
L = 16

CHUNK = 128
NG = 80
TE = NG * CHUNK
E_PAD = TE * NW
RN = R * N
RN_SZ = 80128
CSLICE = RN_SZ // NS
N_PAD = 10240
NROW = N_PAD // NS
ZROW = 128

_mesh = plsc.VectorSubcoreMesh(core_axis_name="c", subcore_axis_name="s")
_sc_params = pltpu.CompilerParams(
    needs_layout_passes=False, use_tc_tiling_on_sc=False)


def _fori(lo, hi, body, unroll=1):
    lax.fori_loop(lo, hi, lambda i, c: (body(i), c)[1], 0, unroll=unroll)


@functools.partial(
    pl.kernel,
    out_type=jax.ShapeDtypeStruct((NC * RN_SZ,), jnp.float32),
    mesh=_mesh,
    compiler_params=_sc_params,
    scratch_types=[
        pltpu.VMEM((NG, CHUNK), jnp.int32),
        pltpu.VMEM((CHUNK,), jnp.float32),
        pltpu.VMEM((CSLICE,), jnp.float32),
        pltpu.VMEM_SHARED((RN_SZ,), jnp.float32),
        pltpu.SemaphoreType.DMA,
    ],
)
def _counts_kernel(cidx_hbm, out_hbm, cidx_all, ones_v, zbuf, counts_sh, sem):
    c = lax.axis_index("c")
    s = lax.axis_index("s")
    wid = c * NS + s
    zero16 = jnp.zeros((L,), jnp.float32)
    one16 = jnp.ones((L,), jnp.float32)
    _fori(0, CSLICE // L, lambda i: zbuf.__setitem__(pl.ds(i * L, L), zero16))
    _fori(0, CHUNK // L, lambda i: ones_v.__setitem__(pl.ds(i * L, L), one16))
    pltpu.sync_copy(cidx_hbm.at[wid], cidx_all)
    pltpu.sync_copy(zbuf, counts_sh.at[pl.ds(s * CSLICE, CSLICE)])
    plsc.subcore_barrier()

    def fire(i):
        pltpu.async_copy(ones_v, counts_sh.at[cidx_all.at[i]], sem, add=True)

    def drain(i):
        pltpu.make_async_copy(ones_v, counts_sh.at[cidx_all.at[0]], sem).wait()

    _fori(0, NG, fire)
    _fori(0, NG, drain)
    plsc.subcore_barrier()
    pltpu.sync_copy(counts_sh.at[pl.ds(s * CSLICE, CSLICE)], zbuf)
    pltpu.sync_copy(zbuf, out_hbm.at[pl.ds(c * RN_SZ + s * CSLICE, CSLICE)])


@functools.partial(
    pl.kernel,
    out_type=jax.ShapeDtypeStruct((E_PAD,), jnp.float32),
    mesh=_mesh,
    compiler_params=_sc_params,
    scratch_types=[
        pltpu.VMEM((RN_SZ,), jnp.float32),
        pltpu.VMEM((TE,), jnp.int32),
        pltpu.VMEM((TE,), jnp.float32),
    ],
)
def _weights_kernel(recip_hbm, cidx_hbm, w_hbm, recip_t, cidx_all, w_all):
    c = lax.axis_index("c")
    s = lax.axis_index("s")
    wid = c * NS + s
    pltpu.sync_copy(recip_hbm, recip_t)
    pltpu.sync_copy(cidx_hbm.at[wid], cidx_all)
    iota = lax.iota(jnp.int32, L)

    def step(i):
        civ = cidx_all[pl.ds(i * L, L)]
        w = plsc.load_gather(recip_t, [civ])
        eid = wid * TE + i * L + iota
        w_all[pl.ds(i * L, L)] = jnp.where(eid < E, w, 0.0)

    _fori(0, TE // L, step, unroll=4)
    pltpu.sync_copy(w_all, w_hbm.at[pl.ds(wid * TE, TE)])


def _make_msg_kernel(out_ch):
    @functools.partial(
        pl.kernel,
        out_type=jax.ShapeDtypeStruct((NC * N_PAD, out_ch), jnp.float32),
        mesh=_mesh,
        compiler_params=_sc_params,
        scratch_types=[
            pltpu.VMEM((NG, CHUNK), jnp.int32),
            pltpu.VMEM((NG, CHUNK), jnp.int32),
            pltpu.VMEM((TE,), jnp.float32),
            pltpu.VMEM((2, CHUNK, out_ch), jnp.float32),
            pltpu.VMEM((ZROW, out_ch), jnp.float32),
            pltpu.VMEM_SHARED((N_PAD, out_ch), jnp.float32),
            pltpu.SemaphoreType.DMA((2,)),
        ],
    )
    def _msg_kernel(hx_hbm, gidx_hbm, dst_hbm, w_hbm, out_hbm,
                    gidx_all, dst_all, w_all, rows_v, zbuf, acc, sem):
        c = lax.axis_index("c")
        s = lax.axis_index("s")
        wid = c * NS + s
        zero16 = jnp.zeros((L,), jnp.float32)

        def zrow(i):
            for t in range(out_ch // L):
                zbuf[i, pl.ds(t * L, L)] = zero16

        _fori(0, ZROW, zrow)
        pltpu.sync_copy(gidx_hbm.at[wid], gidx_all)
        pltpu.sync_copy(dst_hbm.at[wid], dst_all)
        pltpu.sync_copy(w_hbm.at[wid], w_all)
        for k in range(NROW // ZROW):
            pltpu.sync_copy(zbuf, acc.at[pl.ds(s * NROW + k * ZROW, ZROW)])
        plsc.subcore_barrier()

        pltpu.async_copy(hx_hbm.at[gidx_all.at[0]], rows_v.at[0], sem.at[0])

        def pair(g):
            for b in (0, 1):
                cc = 2 * g + b

                @pl.when(cc + 1 < NG)
                def _():
                    pltpu.async_copy(hx_hbm.at[gidx_all.at[cc + 1]],
                                     rows_v.at[b ^ 1], sem.at[b ^ 1])

                pltpu.make_async_copy(hx_hbm.at[gidx_all.at[cc]],
                                      rows_v.at[b], sem.at[b]).wait()

                def srow(j):
                    wb = plsc.load_gather(
                        w_all, [jnp.full((L,), cc * CHUNK + j, jnp.int32)])
                    for t in range(out_ch // L):
                        sl = pl.ds(t * L, L)
                        rows_v[b, j, sl] = rows_v[b, j, sl] * wb

                _fori(0, CHUNK, srow, unroll=2)
                pltpu.sync_copy(rows_v.at[b], acc.at[dst_all.at[cc]], add=True)

        _fori(0, NG // 2, pair)
        plsc.subcore_barrier()
        for k in range(NROW // ZROW):
            sl = pl.ds(s * NROW + k * ZROW, ZROW)
            pltpu.sync_copy(acc.at[sl], zbuf)
            pltpu.sync_copy(zbuf, out_hbm.at[pl.ds(c * N_PAD + s * NROW + k * ZROW, ZROW)])

    return _msg_kernel


_msg_kernel64 = _make_msg_kernel(HID)


_BN = 1000


def _mm1_body(comp_ref, basis_ref, root_ref, bias_ref, x_ref, hx_ref, self_ref):
    xb = x_ref[...]
    ws = []
    for r in range(R):
        w = comp_ref[r, 0] * basis_ref[0]
        for b in range(1, 4):
            w = w + comp_ref[r, b] * basis_ref[b]
        ws.append(w)
    wcat = jnp.concatenate(ws, axis=1)
    hx_ref[...] = jnp.dot(xb, wcat, preferred_element_type=jnp.float32)
    self_ref[...] = (jnp.dot(xb, root_ref[...], preferred_element_type=jnp.float32)
                     + bias_ref[...])


def _mm2_body(comp_ref, basis_ref, root_ref, bias_ref, self1_ref, p0_ref, p1_ref,
              hxa_ref, hxb_ref, self_ref):
    h = jnp.maximum(self1_ref[...] + p0_ref[...] + p1_ref[...], 0.0)
    ws = []
    for r in range(R):
        w = comp_ref[r, 0] * basis_ref[0]
        for b in range(1, 4):
            w = w + comp_ref[r, b] * basis_ref[b]
        ws.append(w)
    wa = jnp.concatenate([w[:, :HID] for w in ws], axis=1)
    wb = jnp.concatenate([w[:, HID:] for w in ws], axis=1)
    hxa_ref[...] = jnp.dot(h, wa, preferred_element_type=jnp.float32)
    hxb_ref[...] = jnp.dot(h, wb, preferred_element_type=jnp.float32)
    self_ref[...] = (jnp.dot(h, root_ref[...], preferred_element_type=jnp.float32)
                     + bias_ref[...])


def _recip_body(p_ref, o_ref):
    o_ref[...] = 1.0 / jnp.maximum(p_ref[0] + p_ref[1], 1.0)


def _final_body(s_ref, qa0_ref, qa1_ref, qb0_ref, qb1_ref, o_ref):
    qa = qa0_ref[...] + qa1_ref[...]
    qb = qb0_ref[...] + qb1_ref[...]
    o_ref[...] = s_ref[...] + jnp.concatenate([qa, qb], axis=1)


def _full_spec(shape):
    nd = len(shape)
    return pl.BlockSpec(shape, lambda i, _n=nd: (0,) * _n)


def _row_spec(cols):
    return pl.BlockSpec((_BN, cols), lambda i: (i, 0))


def _mm1(x, comp, basis, root, bias):
    return pl.pallas_call(
        _mm1_body,
        grid=(N // _BN,),
        in_specs=[
            pl.BlockSpec(memory_space=pltpu.SMEM),
            _full_spec((4, IN_CH, HID)),
            _full_spec((IN_CH, HID)),
            _full_spec((1, HID)),
            _row_spec(IN_CH),
        ],
        out_specs=[_row_spec(R * HID), _row_spec(HID)],
        out_shape=[
            jax.ShapeDtypeStruct((N, R * HID), jnp.float32),
            jax.ShapeDtypeStruct((N, HID), jnp.float32),
        ],
    )(comp, basis, root, bias.reshape(1, HID), x)


def _mm2(self1, p0, p1, comp, basis, root, bias):
    return pl.pallas_call(
        _mm2_body,
        grid=(N // _BN,),
        in_specs=[
            pl.BlockSpec(memory_space=pltpu.SMEM),
            _full_spec((4, HID, OUT_CH)),
            _full_spec((HID, OUT_CH)),
            _full_spec((1, OUT_CH)),
            _row_spec(HID),
            _row_spec(HID),
            _row_spec(HID),
        ],
        out_specs=[_row_spec(R * HID), _row_spec(R * HID), _row_spec(OUT_CH)],
        out_shape=[
            jax.ShapeDtypeStruct((N, R * HID), jnp.float32),
            jax.ShapeDtypeStruct((N, R * HID), jnp.float32),
            jax.ShapeDtypeStruct((N, OUT_CH), jnp.float32),
        ],
    )(comp, basis, root, bias.reshape(1, OUT_CH), self1, p0, p1)


def _recip(parts):
    return pl.pallas_call(
        _recip_body,
        grid=(1,),
        in_specs=[_full_spec((2, RN_SZ // 128, 128))],
        out_specs=_full_spec((RN_SZ // 128, 128)),
        out_shape=jax.ShapeDtypeStruct((RN_SZ // 128, 128), jnp.float32),
    )(parts.reshape(2, RN_SZ // 128, 128))


def _final(s, qa0, qa1, qb0, qb1):
    return pl.pallas_call(
        _final_body,
        grid=(N // _BN,),
        in_specs=[_row_spec(OUT_CH)] + [_row_spec(HID)] * 4,
        out_specs=_row_spec(OUT_CH),
        out_shape=jax.ShapeDtypeStruct((N, OUT_CH), jnp.float32),
    )(s, qa0, qa1, qb0, qb1)


def kernel(x, edge_index, edge_type, comp1, basis1, root1, bias1,
           comp2, basis2, root2, bias2):
    src = edge_index[0].astype(jnp.int32)
    dst = edge_index[1].astype(jnp.int32)
    et = edge_type.astype(jnp.int32)
    gidx = src * R + et
    cidx = dst * R + et
    pad = E_PAD - E
    gidx_p = jnp.concatenate([gidx, jnp.zeros((pad,), jnp.int32)])
    dst_p = jnp.concatenate([dst, jnp.zeros((pad,), jnp.int32)])
    cidx_p = jnp.concatenate([cidx, jnp.full((pad,), RN, jnp.int32)])

    parts = _counts_kernel(cidx_p.reshape(NW, NG, CHUNK))
    recip = _recip(parts).reshape(RN_SZ)
    w = _weights_kernel(recip, cidx_p.reshape(NW, TE))
    w2 = w.reshape(NW, TE)
    gidx3 = gidx_p.reshape(NW, NG, CHUNK)
    dst3 = dst_p.reshape(NW, NG, CHUNK)

    hx1, self1 = _mm1(x, comp1, basis1, root1, bias1)
    p = _msg_kernel64(hx1.reshape(N * R, HID), gidx3, dst3, w2)
    hx2a, hx2b, self2 = _mm2(self1, p[:N], p[N_PAD:N_PAD + N],
                             comp2, basis2, root2, bias2)
    qa = _msg_kernel64(hx2a.reshape(N * R, HID), gidx3, dst3, w2)
    qb = _msg_kernel64(hx2b.reshape(N * R, HID), gidx3, dst3, w2)
    return _final(self2, qa[:N], qa[N_PAD:N_PAD + N],
                  qb[:N], qb[N_PAD:N_PAD + N])


# speedup vs baseline: 24.7877x; 1.0036x over previous
import functools
import jax
import jax.numpy as jnp
from jax import lax
from jax.experimental import pallas as pl
from jax.experimental.pallas import tpu as pltpu
from jax.experimental.pallas import tpu_sc as plsc

N = 10000
E = 320000
IN_CH = 128
HID = 64
OUT_CH = 128
R = 8

NC = 2
NS = 16
NW = NC * NS
L = 16

CHUNK = 128
NG = 80
TE = NG * CHUNK
E_PAD = TE * NW
RN = R * N
RN_SZ = 80128
CSLICE = RN_SZ // NS
N_PAD = 10240
NROW = N_PAD // NS
ZROW = 128

_mesh = plsc.VectorSubcoreMesh(core_axis_name="c", subcore_axis_name="s")
_sc_params = pltpu.CompilerParams(
    needs_layout_passes=False, use_tc_tiling_on_sc=False)


def _fori(lo, hi, body, unroll=1):
    lax.fori_loop(lo, hi, lambda i, c: (body(i), c)[1], 0, unroll=unroll)


@functools.partial(
    pl.kernel,
    out_type=jax.ShapeDtypeStruct((NC * RN_SZ,), jnp.float32),
    mesh=_mesh,
    compiler_params=_sc_params,
    scratch_types=[
        pltpu.VMEM((NG, CHUNK), jnp.int32),
        pltpu.VMEM((CHUNK,), jnp.float32),
        pltpu.VMEM((CSLICE,), jnp.float32),
        pltpu.VMEM_SHARED((RN_SZ,), jnp.float32),
        pltpu.SemaphoreType.DMA,
    ],
)
def _counts_kernel(cidx_hbm, out_hbm, cidx_all, ones_v, zbuf, counts_sh, sem):
    c = lax.axis_index("c")
    s = lax.axis_index("s")
    wid = c * NS + s
    zero16 = jnp.zeros((L,), jnp.float32)
    one16 = jnp.ones((L,), jnp.float32)
    _fori(0, CSLICE // L, lambda i: zbuf.__setitem__(pl.ds(i * L, L), zero16))
    _fori(0, CHUNK // L, lambda i: ones_v.__setitem__(pl.ds(i * L, L), one16))
    pltpu.sync_copy(cidx_hbm.at[wid], cidx_all)
    pltpu.sync_copy(zbuf, counts_sh.at[pl.ds(s * CSLICE, CSLICE)])
    plsc.subcore_barrier()

    def fire(i):
        pltpu.async_copy(ones_v, counts_sh.at[cidx_all.at[i]], sem, add=True)

    def drain(i):
        pltpu.make_async_copy(ones_v, counts_sh.at[cidx_all.at[0]], sem).wait()

    _fori(0, NG, fire)
    _fori(0, NG, drain)
    plsc.subcore_barrier()
    pltpu.sync_copy(counts_sh.at[pl.ds(s * CSLICE, CSLICE)], zbuf)
    pltpu.sync_copy(zbuf, out_hbm.at[pl.ds(c * RN_SZ + s * CSLICE, CSLICE)])


@functools.partial(
    pl.kernel,
    out_type=jax.ShapeDtypeStruct((E_PAD,), jnp.float32),
    mesh=_mesh,
    compiler_params=_sc_params,
    scratch_types=[
        pltpu.VMEM((RN_SZ,), jnp.float32),
        pltpu.VMEM((TE,), jnp.int32),
        pltpu.VMEM((TE,), jnp.float32),
    ],
)
def _weights_kernel(recip_hbm, cidx_hbm, w_hbm, recip_t, cidx_all, w_all):
    c = lax.axis_index("c")
    s = lax.axis_index("s")
    wid = c * NS + s
    pltpu.sync_copy(recip_hbm, recip_t)
    pltpu.sync_copy(cidx_hbm.at[wid], cidx_all)
    iota = lax.iota(jnp.int32, L)

    def step(i):
        civ = cidx_all[pl.ds(i * L, L)]
        w = plsc.load_gather(recip_t, [civ])
        eid = wid * TE + i * L + iota
        w_all[pl.ds(i * L, L)] = jnp.where(eid < E, w, 0.0)

    _fori(0, TE // L, step, unroll=4)
    pltpu.sync_copy(w_all, w_hbm.at[pl.ds(wid * TE, TE)])


def _make_msg_kernel(out_ch):
    @functools.partial(
        pl.kernel,
        out_type=jax.ShapeDtypeStruct((NC * N_PAD, out_ch), jnp.float32),
        mesh=_mesh,
        compiler_params=_sc_params,
        scratch_types=[
            pltpu.VMEM((NG, CHUNK), jnp.int32),
            pltpu.VMEM((NG, CHUNK), jnp.int32),
            pltpu.VMEM((TE,), jnp.float32),
            pltpu.VMEM((4, CHUNK, out_ch), jnp.float32),
            pltpu.VMEM((ZROW, out_ch), jnp.float32),
            pltpu.VMEM_SHARED((N_PAD, out_ch), jnp.float32),
            pltpu.SemaphoreType.DMA((4,)),
            pltpu.SemaphoreType.DMA((4,)),
        ],
    )
    def _msg_kernel(hx_hbm, gidx_hbm, dst_hbm, w_hbm, out_hbm,
                    gidx_all, dst_all, w_all, rows_v, zbuf, acc, gsem, ssem):
        c = lax.axis_index("c")
        s = lax.axis_index("s")
        wid = c * NS + s
        zero16 = jnp.zeros((L,), jnp.float32)

        def zrow(i):
            for t in range(out_ch // L):
                zbuf[i, pl.ds(t * L, L)] = zero16

        _fori(0, ZROW, zrow)
        pltpu.sync_copy(gidx_hbm.at[wid], gidx_all)
        pltpu.sync_copy(dst_hbm.at[wid], dst_all)
        pltpu.sync_copy(w_hbm.at[wid], w_all)
        for k in range(NROW // ZROW):
            pltpu.sync_copy(zbuf, acc.at[pl.ds(s * NROW + k * ZROW, ZROW)])
        plsc.subcore_barrier()

        pltpu.async_copy(hx_hbm.at[gidx_all.at[0]], rows_v.at[0], gsem.at[0])
        pltpu.async_copy(hx_hbm.at[gidx_all.at[1]], rows_v.at[1], gsem.at[1])

        def quad(g):
            for b in range(4):
                cc = 4 * g + b
                bb = (b + 2) & 3

                @pl.when(jnp.logical_and(cc + 2 < NG, cc >= 2))
                def _():
                    pltpu.make_async_copy(
                        rows_v.at[bb], acc.at[dst_all.at[cc - 2]],
                        ssem.at[bb]).wait()

                @pl.when(cc + 2 < NG)
                def _():
                    pltpu.async_copy(hx_hbm.at[gidx_all.at[cc + 2]],
                                     rows_v.at[bb], gsem.at[bb])

                pltpu.make_async_copy(hx_hbm.at[gidx_all.at[cc]],
                                      rows_v.at[b], gsem.at[b]).wait()

                def srow(j):
                    wb = plsc.load_gather(
                        w_all, [jnp.full((L,), cc * CHUNK + j, jnp.int32)])
                    for t in range(out_ch // L):
                        sl = pl.ds(t * L, L)
                        rows_v[b, j, sl] = rows_v[b, j, sl] * wb

                _fori(0, CHUNK, srow, unroll=2)
                pltpu.async_copy(rows_v.at[b], acc.at[dst_all.at[cc]],
                                 ssem.at[b], add=True)

        _fori(0, NG // 4, quad)
        for b in range(4):
            pltpu.make_async_copy(rows_v.at[b], acc.at[dst_all.at[NG - 4 + b]],
                                  ssem.at[b]).wait()
        plsc.subcore_barrier()
        for k in range(NROW // ZROW):
            sl = pl.ds(s * NROW + k * ZROW, ZROW)
            pltpu.sync_copy(acc.at[sl], zbuf)
            pltpu.sync_copy(zbuf, out_hbm.at[pl.ds(c * N_PAD + s * NROW + k * ZROW, ZROW)])

    return _msg_kernel


_msg_kernel64 = _make_msg_kernel(HID)


_BN = 1000


def _mm1_body(comp_ref, basis_ref, root_ref, bias_ref, x_ref, hx_ref, self_ref):
    xb = x_ref[...]
    ws = []
    for r in range(R):
        w = comp_ref[r, 0] * basis_ref[0]
        for b in range(1, 4):
            w = w + comp_ref[r, b] * basis_ref[b]
        ws.append(w)
    wcat = jnp.concatenate(ws, axis=1)
    hx_ref[...] = jnp.dot(xb, wcat, preferred_element_type=jnp.float32)
    self_ref[...] = (jnp.dot(xb, root_ref[...], preferred_element_type=jnp.float32)
                     + bias_ref[...])


def _mm2_body(comp_ref, basis_ref, root_ref, bias_ref, self1_ref, p0_ref, p1_ref,
              hxa_ref, hxb_ref, self_ref):
    h = jnp.maximum(self1_ref[...] + p0_ref[...] + p1_ref[...], 0.0)
    ws = []
    for r in range(R):
        w = comp_ref[r, 0] * basis_ref[0]
        for b in range(1, 4):
            w = w + comp_ref[r, b] * basis_ref[b]
        ws.append(w)
    wa = jnp.concatenate([w[:, :HID] for w in ws], axis=1)
    wb = jnp.concatenate([w[:, HID:] for w in ws], axis=1)
    hxa_ref[...] = jnp.dot(h, wa, preferred_element_type=jnp.float32)
    hxb_ref[...] = jnp.dot(h, wb, preferred_element_type=jnp.float32)
    self_ref[...] = (jnp.dot(h, root_ref[...], preferred_element_type=jnp.float32)
                     + bias_ref[...])


def _recip_body(p_ref, o_ref):
    o_ref[...] = 1.0 / jnp.maximum(p_ref[0] + p_ref[1], 1.0)


def _final_body(s_ref, qa0_ref, qa1_ref, qb0_ref, qb1_ref, o_ref):
    qa = qa0_ref[...] + qa1_ref[...]
    qb = qb0_ref[...] + qb1_ref[...]
    o_ref[...] = s_ref[...] + jnp.concatenate([qa, qb], axis=1)


def _full_spec(shape):
    nd = len(shape)
    return pl.BlockSpec(shape, lambda i, _n=nd: (0,) * _n)


def _row_spec(cols):
    return pl.BlockSpec((_BN, cols), lambda i: (i, 0))


def _mm1(x, comp, basis, root, bias):
    return pl.pallas_call(
        _mm1_body,
        grid=(N // _BN,),
        in_specs=[
            pl.BlockSpec(memory_space=pltpu.SMEM),
            _full_spec((4, IN_CH, HID)),
            _full_spec((IN_CH, HID)),
            _full_spec((1, HID)),
            _row_spec(IN_CH),
        ],
        out_specs=[_row_spec(R * HID), _row_spec(HID)],
        out_shape=[
            jax.ShapeDtypeStruct((N, R * HID), jnp.float32),
            jax.ShapeDtypeStruct((N, HID), jnp.float32),
        ],
    )(comp, basis, root, bias.reshape(1, HID), x)


def _mm2(self1, p0, p1, comp, basis, root, bias):
    return pl.pallas_call(
        _mm2_body,
        grid=(N // _BN,),
        in_specs=[
            pl.BlockSpec(memory_space=pltpu.SMEM),
            _full_spec((4, HID, OUT_CH)),
            _full_spec((HID, OUT_CH)),
            _full_spec((1, OUT_CH)),
            _row_spec(HID),
            _row_spec(HID),
            _row_spec(HID),
        ],
        out_specs=[_row_spec(R * HID), _row_spec(R * HID), _row_spec(OUT_CH)],
        out_shape=[
            jax.ShapeDtypeStruct((N, R * HID), jnp.float32),
            jax.ShapeDtypeStruct((N, R * HID), jnp.float32),
            jax.ShapeDtypeStruct((N, OUT_CH), jnp.float32),
        ],
    )(comp, basis, root, bias.reshape(1, OUT_CH), self1, p0, p1)


def _recip(parts):
    return pl.pallas_call(
        _recip_body,
        grid=(1,),
        in_specs=[_full_spec((2, RN_SZ // 128, 128))],
        out_specs=_full_spec((RN_SZ // 128, 128)),
        out_shape=jax.ShapeDtypeStruct((RN_SZ // 128, 128), jnp.float32),
    )(parts.reshape(2, RN_SZ // 128, 128))


def _final(s, qa0, qa1, qb0, qb1):
    return pl.pallas_call(
        _final_body,
        grid=(N // _BN,),
        in_specs=[_row_spec(OUT_CH)] + [_row_spec(HID)] * 4,
        out_specs=_row_spec(OUT_CH),
        out_shape=jax.ShapeDtypeStruct((N, OUT_CH), jnp.float32),
    )(s, qa0, qa1, qb0, qb1)


def kernel(x, edge_index, edge_type, comp1, basis1, root1, bias1,
           comp2, basis2, root2, bias2):
    src = edge_index[0].astype(jnp.int32)
    dst = edge_index[1].astype(jnp.int32)
    et = edge_type.astype(jnp.int32)
    gidx = src * R + et
    cidx = dst * R + et
    pad = E_PAD - E
    gidx_p = jnp.concatenate([gidx, jnp.zeros((pad,), jnp.int32)])
    dst_p = jnp.concatenate([dst, jnp.zeros((pad,), jnp.int32)])
    cidx_p = jnp.concatenate([cidx, jnp.full((pad,), RN, jnp.int32)])

    parts = _counts_kernel(cidx_p.reshape(NW, NG, CHUNK))
    recip = _recip(parts).reshape(RN_SZ)
    w = _weights_kernel(recip, cidx_p.reshape(NW, TE))
    w2 = w.reshape(NW, TE)
    gidx3 = gidx_p.reshape(NW, NG, CHUNK)
    dst3 = dst_p.reshape(NW, NG, CHUNK)

    hx1, self1 = _mm1(x, comp1, basis1, root1, bias1)
    p = _msg_kernel64(hx1.reshape(N * R, HID), gidx3, dst3, w2)
    hx2a, hx2b, self2 = _mm2(self1, p[:N], p[N_PAD:N_PAD + N],
                             comp2, basis2, root2, bias2)
    qa = _msg_kernel64(hx2a.reshape(N * R, HID), gidx3, dst3, w2)
    qb = _msg_kernel64(hx2b.reshape(N * R, HID), gidx3, dst3, w2)
    return _final(self2, qa[:N], qa[N_PAD:N_PAD + N],
                  qb[:N], qb[N_PAD:N_PAD + N])


# speedup vs baseline: 25.6410x; 1.0344x over previous
import functools
import jax
import jax.numpy as jnp
from jax import lax
from jax.experimental import pallas as pl
from jax.experimental.pallas import tpu as pltpu
from jax.experimental.pallas import tpu_sc as plsc

N = 10000
E = 320000
IN_CH = 128
HID = 64
OUT_CH = 128
R = 8

NC = 2
NS = 16
NW = NC * NS
L = 16

CHUNK = 128
NG = 80
TE = NG * CHUNK
E_PAD = TE * NW
RN = R * N
RN_SZ = 80128
CSLICE = RN_SZ // NS
N_PAD = 10240
NROW = N_PAD // NS
ZROW = 128

G0 = 108
G1 = 52
EA_CH = 16 * G0 + 16 * G1 + (G0 - G1)
M_PAD = (EA_CH * CHUNK) - E_PAD

_mesh = plsc.VectorSubcoreMesh(core_axis_name="c", subcore_axis_name="s")
_sc_params = pltpu.CompilerParams(
    needs_layout_passes=False, use_tc_tiling_on_sc=False)


def _fori(lo, hi, body, unroll=1):
    lax.fori_loop(lo, hi, lambda i, c: (body(i), c)[1], 0, unroll=unroll)


@functools.partial(
    pl.kernel,
    out_type=jax.ShapeDtypeStruct((NC * RN_SZ,), jnp.float32),
    mesh=_mesh,
    compiler_params=_sc_params,
    scratch_types=[
        pltpu.VMEM((NG, CHUNK), jnp.int32),
        pltpu.VMEM((CHUNK,), jnp.float32),
        pltpu.VMEM((CSLICE,), jnp.float32),
        pltpu.VMEM_SHARED((RN_SZ,), jnp.float32),
        pltpu.SemaphoreType.DMA,
    ],
)
def _counts_kernel(cidx_hbm, out_hbm, cidx_all, ones_v, zbuf, counts_sh, sem):
    c = lax.axis_index("c")
    s = lax.axis_index("s")
    wid = c * NS + s
    zero16 = jnp.zeros((L,), jnp.float32)
    one16 = jnp.ones((L,), jnp.float32)
    _fori(0, CSLICE // L, lambda i: zbuf.__setitem__(pl.ds(i * L, L), zero16))
    _fori(0, CHUNK // L, lambda i: ones_v.__setitem__(pl.ds(i * L, L), one16))
    pltpu.sync_copy(cidx_hbm.at[wid], cidx_all)
    pltpu.sync_copy(zbuf, counts_sh.at[pl.ds(s * CSLICE, CSLICE)])
    plsc.subcore_barrier()

    def fire(i):
        pltpu.async_copy(ones_v, counts_sh.at[cidx_all.at[i]], sem, add=True)

    def drain(i):
        pltpu.make_async_copy(ones_v, counts_sh.at[cidx_all.at[0]], sem).wait()

    _fori(0, NG, fire)
    _fori(0, NG, drain)
    plsc.subcore_barrier()
    pltpu.sync_copy(counts_sh.at[pl.ds(s * CSLICE, CSLICE)], zbuf)
    pltpu.sync_copy(zbuf, out_hbm.at[pl.ds(c * RN_SZ + s * CSLICE, CSLICE)])


@functools.partial(
    pl.kernel,
    out_type=jax.ShapeDtypeStruct((E_PAD,), jnp.float32),
    mesh=_mesh,
    compiler_params=_sc_params,
    scratch_types=[
        pltpu.VMEM((RN_SZ,), jnp.float32),
        pltpu.VMEM((TE,), jnp.int32),
        pltpu.VMEM((TE,), jnp.float32),
    ],
)
def _weights_kernel(recip_hbm, cidx_hbm, w_hbm, recip_t, cidx_all, w_all):
    c = lax.axis_index("c")
    s = lax.axis_index("s")
    wid = c * NS + s
    pltpu.sync_copy(recip_hbm, recip_t)
    pltpu.sync_copy(cidx_hbm.at[wid], cidx_all)
    iota = lax.iota(jnp.int32, L)

    def step(i):
        civ = cidx_all[pl.ds(i * L, L)]
        w = plsc.load_gather(recip_t, [civ])
        eid = wid * TE + i * L + iota
        w_all[pl.ds(i * L, L)] = jnp.where(eid < E, w, 0.0)

    _fori(0, TE // L, step, unroll=4)
    pltpu.sync_copy(w_all, w_hbm.at[pl.ds(wid * TE, TE)])


def _make_msg_kernel(out_ch):
    @functools.partial(
        pl.kernel,
        out_type=jax.ShapeDtypeStruct((NC * N_PAD, out_ch), jnp.float32),
        mesh=_mesh,
        compiler_params=_sc_params,
        scratch_types=[
            pltpu.VMEM((G0, CHUNK), jnp.int32),
            pltpu.VMEM((G0, CHUNK), jnp.int32),
            pltpu.VMEM((G0, CHUNK), jnp.float32),
            pltpu.VMEM((4, CHUNK, out_ch), jnp.float32),
            pltpu.VMEM((ZROW, out_ch), jnp.float32),
            pltpu.VMEM_SHARED((N_PAD, out_ch), jnp.float32),
            pltpu.SemaphoreType.DMA((4,)),
            pltpu.SemaphoreType.DMA((4,)),
        ],
    )
    def _msg_kernel(hx_hbm, gidx_hbm, dst_hbm, w_hbm, out_hbm,
                    gidx_all, dst_all, w_all, rows_v, zbuf, acc, gsem, ssem):
        c = lax.axis_index("c")
        s = lax.axis_index("s")
        base = jnp.where(c == 0, s * G0, NS * G0 + s * G1)
        ng = jnp.where(c == 0, G0, G1)
        zero16 = jnp.zeros((L,), jnp.float32)

        def zrow(i):
            for t in range(out_ch // L):
                zbuf[i, pl.ds(t * L, L)] = zero16

        _fori(0, ZROW, zrow)
        pltpu.sync_copy(gidx_hbm.at[pl.ds(base, G0)], gidx_all)
        pltpu.sync_copy(dst_hbm.at[pl.ds(base, G0)], dst_all)
        pltpu.sync_copy(w_hbm.at[pl.ds(base, G0)], w_all)
        for k in range(NROW // ZROW):
            pltpu.sync_copy(zbuf, acc.at[pl.ds(s * NROW + k * ZROW, ZROW)])
        plsc.subcore_barrier()

        pltpu.async_copy(hx_hbm.at[gidx_all.at[0]], rows_v.at[0], gsem.at[0])
        pltpu.async_copy(hx_hbm.at[gidx_all.at[1]], rows_v.at[1], gsem.at[1])

        def quad(g):
            for b in range(4):
                cc = 4 * g + b
                bb = (b + 2) & 3

                @pl.when(jnp.logical_and(cc + 2 < ng, cc >= 2))
                def _():
                    pltpu.make_async_copy(
                        rows_v.at[bb], acc.at[dst_all.at[cc - 2]],
                        ssem.at[bb]).wait()

                @pl.when(cc + 2 < ng)
                def _():
                    pltpu.async_copy(hx_hbm.at[gidx_all.at[cc + 2]],
                                     rows_v.at[bb], gsem.at[bb])

                pltpu.make_async_copy(hx_hbm.at[gidx_all.at[cc]],
                                      rows_v.at[b], gsem.at[b]).wait()

                def srow(j):
                    wb = plsc.load_gather(
                        w_all, [jnp.full((L,), cc, jnp.int32),
                                jnp.full((L,), j, jnp.int32)])
                    for t in range(out_ch // L):
                        sl = pl.ds(t * L, L)
                        rows_v[b, j, sl] = rows_v[b, j, sl] * wb

                _fori(0, CHUNK, srow, unroll=2)
                pltpu.async_copy(rows_v.at[b], acc.at[dst_all.at[cc]],
                                 ssem.at[b], add=True)

        _fori(0, ng // 4, quad)
        for b in range(4):
            pltpu.make_async_copy(rows_v.at[b], acc.at[dst_all.at[ng - 4 + b]],
                                  ssem.at[b]).wait()
        plsc.subcore_barrier()
        for k in range(NROW // ZROW):
            sl = pl.ds(s * NROW + k * ZROW, ZROW)
            pltpu.sync_copy(acc.at[sl], zbuf)
            pltpu.sync_copy(zbuf, out_hbm.at[pl.ds(c * N_PAD + s * NROW + k * ZROW, ZROW)])

    return _msg_kernel


_msg_kernel64 = _make_msg_kernel(HID)


_BN = 1000


def _mm1_body(comp_ref, basis_ref, root_ref, bias_ref, x_ref, hx_ref, self_ref):
    xb = x_ref[...]
    ws = []
    for r in range(R):
        w = comp_ref[r, 0] * basis_ref[0]
        for b in range(1, 4):
            w = w + comp_ref[r, b] * basis_ref[b]
        ws.append(w)
    wcat = jnp.concatenate(ws, axis=1)
    hx_ref[...] = jnp.dot(xb, wcat, preferred_element_type=jnp.float32)
    self_ref[...] = (jnp.dot(xb, root_ref[...], preferred_element_type=jnp.float32)
                     + bias_ref[...])


def _mm2_body(comp_ref, basis_ref, root_ref, bias_ref, self1_ref, p0_ref, p1_ref,
              hxa_ref, hxb_ref, self_ref):
    h = jnp.maximum(self1_ref[...] + p0_ref[...] + p1_ref[...], 0.0)
    ws = []
    for r in range(R):
        w = comp_ref[r, 0] * basis_ref[0]
        for b in range(1, 4):
            w = w + comp_ref[r, b] * basis_ref[b]
        ws.append(w)
    wa = jnp.concatenate([w[:, :HID] for w in ws], axis=1)
    wb = jnp.concatenate([w[:, HID:] for w in ws], axis=1)
    hxa_ref[...] = jnp.dot(h, wa, preferred_element_type=jnp.float32)
    hxb_ref[...] = jnp.dot(h, wb, preferred_element_type=jnp.float32)
    self_ref[...] = (jnp.dot(h, root_ref[...], preferred_element_type=jnp.float32)
                     + bias_ref[...])


def _recip_body(p_ref, o_ref):
    o_ref[...] = 1.0 / jnp.maximum(p_ref[0] + p_ref[1], 1.0)


def _final_body(s_ref, qa0_ref, qa1_ref, qb0_ref, qb1_ref, o_ref):
    qa = qa0_ref[...] + qa1_ref[...]
    qb = qb0_ref[...] + qb1_ref[...]
    o_ref[...] = s_ref[...] + jnp.concatenate([qa, qb], axis=1)


def _full_spec(shape):
    nd = len(shape)
    return pl.BlockSpec(shape, lambda i, _n=nd: (0,) * _n)


def _row_spec(cols):
    return pl.BlockSpec((_BN, cols), lambda i: (i, 0))


def _mm1(x, comp, basis, root, bias):
    return pl.pallas_call(
        _mm1_body,
        grid=(N // _BN,),
        in_specs=[
            pl.BlockSpec(memory_space=pltpu.SMEM),
            _full_spec((4, IN_CH, HID)),
            _full_spec((IN_CH, HID)),
            _full_spec((1, HID)),
            _row_spec(IN_CH),
        ],
        out_specs=[_row_spec(R * HID), _row_spec(HID)],
        out_shape=[
            jax.ShapeDtypeStruct((N, R * HID), jnp.float32),
            jax.ShapeDtypeStruct((N, HID), jnp.float32),
        ],
    )(comp, basis, root, bias.reshape(1, HID), x)


def _mm2(self1, p0, p1, comp, basis, root, bias):
    return pl.pallas_call(
        _mm2_body,
        grid=(N // _BN,),
        in_specs=[
            pl.BlockSpec(memory_space=pltpu.SMEM),
            _full_spec((4, HID, OUT_CH)),
            _full_spec((HID, OUT_CH)),
            _full_spec((1, OUT_CH)),
            _row_spec(HID),
            _row_spec(HID),
            _row_spec(HID),
        ],
        out_specs=[_row_spec(R * HID), _row_spec(R * HID), _row_spec(OUT_CH)],
        out_shape=[
            jax.ShapeDtypeStruct((N, R * HID), jnp.float32),
            jax.ShapeDtypeStruct((N, R * HID), jnp.float32),
            jax.ShapeDtypeStruct((N, OUT_CH), jnp.float32),
        ],
    )(comp, basis, root, bias.reshape(1, OUT_CH), self1, p0, p1)


def _recip(parts):
    return pl.pallas_call(
        _recip_body,
        grid=(1,),
        in_specs=[_full_spec((2, RN_SZ // 128, 128))],
        out_specs=_full_spec((RN_SZ // 128, 128)),
        out_shape=jax.ShapeDtypeStruct((RN_SZ // 128, 128), jnp.float32),
    )(parts.reshape(2, RN_SZ // 128, 128))


def _final(s, qa0, qa1, qb0, qb1):
    return pl.pallas_call(
        _final_body,
        grid=(N // _BN,),
        in_specs=[_row_spec(OUT_CH)] + [_row_spec(HID)] * 4,
        out_specs=_row_spec(OUT_CH),
        out_shape=jax.ShapeDtypeStruct((N, OUT_CH), jnp.float32),
    )(s, qa0, qa1, qb0, qb1)


def kernel(x, edge_index, edge_type, comp1, basis1, root1, bias1,
           comp2, basis2, root2, bias2):
    src = edge_index[0].astype(jnp.int32)
    dst = edge_index[1].astype(jnp.int32)
    et = edge_type.astype(jnp.int32)
    gidx = src * R + et
    cidx = dst * R + et
    pad = E_PAD - E
    gidx_p = jnp.concatenate([gidx, jnp.zeros((pad,), jnp.int32)])
    dst_p = jnp.concatenate([dst, jnp.zeros((pad,), jnp.int32)])
    cidx_p = jnp.concatenate([cidx, jnp.full((pad,), RN, jnp.int32)])

    parts = _counts_kernel(cidx_p.reshape(NW, NG, CHUNK))
    recip = _recip(parts).reshape(RN_SZ)
    w = _weights_kernel(recip, cidx_p.reshape(NW, TE))
    zi = jnp.zeros((M_PAD,), jnp.int32)
    w2 = jnp.concatenate([w, jnp.zeros((M_PAD,), jnp.float32)]).reshape(EA_CH, CHUNK)
    gidx3 = jnp.concatenate([gidx_p, zi]).reshape(EA_CH, CHUNK)
    dst3 = jnp.concatenate([dst_p, zi]).reshape(EA_CH, CHUNK)

    hx1, self1 = _mm1(x, comp1, basis1, root1, bias1)
    p = _msg_kernel64(hx1.reshape(N * R, HID), gidx3, dst3, w2)
    hx2a, hx2b, self2 = _mm2(self1, p[:N], p[N_PAD:N_PAD + N],
                             comp2, basis2, root2, bias2)
    qa = _msg_kernel64(hx2a.reshape(N * R, HID), gidx3, dst3, w2)
    qb = _msg_kernel64(hx2b.reshape(N * R, HID), gidx3, dst3, w2)
    return _final(self2, qa[:N], qa[N_PAD:N_PAD + N],
                  qb[:N], qb[N_PAD:N_PAD + N])
